# baseline (speedup 1.0000x reference)
import functools

import jax
import jax.numpy as jnp
from jax import lax
from jax.experimental import pallas as pl
from jax.experimental.pallas import tpu as pltpu
from jax.experimental.pallas import tpu_sc as plsc

N_NODES = 10000
N_EDGES = 320000
D = 128
N_LOC_PAD = 256
BN_EPS = 1e-5

NC, NS, L = 2, 16, 16
NW = NC * NS
EPW = N_EDGES // NW
CH = 128
NCHUNK = EPW // CH
NPAD = 10240
RPS = NPAD // NS

_sc_mesh = plsc.VectorSubcoreMesh(core_axis_name="c", subcore_axis_name="s",
                                  num_cores=NC, num_subcores=NS)
_sc_params = pltpu.CompilerParams(needs_layout_passes=False)


DEG_CH = 2000


@functools.partial(
    pl.kernel,
    out_type=jax.ShapeDtypeStruct((NW, N_NODES), jnp.float32),
    mesh=_sc_mesh,
    compiler_params=_sc_params,
    scratch_types=[pltpu.VMEM((DEG_CH,), jnp.int32),
                   pltpu.VMEM((N_NODES,), jnp.float32)])
def _deg_kernel(dst_hbm, out_hbm, idx_v, hist_v):
    wid = lax.axis_index("s") * NC + lax.axis_index("c")

    def zero_body(i, carry):
        hist_v[pl.ds(i * L, L)] = jnp.zeros((L,), jnp.float32)
        return carry
    lax.fori_loop(0, N_NODES // L, zero_body, 0)

    def chunk_body(c, carry):
        pltpu.sync_copy(dst_hbm.at[pl.ds(wid * EPW + c * DEG_CH, DEG_CH)],
                        idx_v)

        def grp(j, carry2):
            ids = idx_v[pl.ds(j * L, L)]
            plsc.addupdate_scatter(hist_v, [ids], jnp.ones((L,), jnp.float32))
            return carry2
        lax.fori_loop(0, DEG_CH // L, grp, 0)
        return carry
    lax.fori_loop(0, EPW // DEG_CH, chunk_body, 0)
    pltpu.sync_copy(hist_v, out_hbm.at[wid])


EPW_P = 10240
NCHUNK_P = EPW_P // CH
NBUF = 2


@functools.partial(
    pl.kernel,
    out_type=jax.ShapeDtypeStruct((NC, NPAD, D), jnp.float32),
    mesh=_sc_mesh,
    compiler_params=_sc_params,
    scratch_types=[pltpu.VMEM((CH,), jnp.int32)] * (2 * NBUF)
                  + [pltpu.VMEM((CH, D), jnp.float32)] * NBUF
                  + [pltpu.VMEM_SHARED((NPAD, D), jnp.float32)]
                  + [pltpu.SemaphoreType.DMA] * (3 * NBUF))
def _agg_kernel(m_hbm, src_hbm, dst_hbm, out_hbm, *rest):
    sidx = rest[0:NBUF]
    didx = rest[NBUF:2 * NBUF]
    rows = rest[2 * NBUF:3 * NBUF]
    acc_sh = rest[3 * NBUF]
    isem = rest[3 * NBUF + 1:3 * NBUF + 1 + 2 * NBUF]
    gsem = rest[3 * NBUF + 1 + 2 * NBUF:]
    cid = lax.axis_index("c")
    sid = lax.axis_index("s")
    wid = sid * NC + cid

    def zb(i, carry):
        r = i // (D // L)
        col = (i % (D // L)) * L
        rows[0][r, pl.ds(col, L)] = jnp.zeros((L,), jnp.float32)
        return carry
    lax.fori_loop(0, CH * (D // L), zb, 0)

    def zcopy(k, carry):
        pltpu.sync_copy(rows[0], acc_sh.at[pl.ds(sid * RPS + k * CH, CH)])
        return carry
    lax.fori_loop(0, RPS // CH, zcopy, 0)
    plsc.subcore_barrier()

    def grp(g, carry):
        base = wid * EPW_P + (2 * g) * CH
        ic = []
        for b in range(NBUF):
            ic.append(pltpu.async_copy(
                src_hbm.at[pl.ds(base + b * CH, CH)], sidx[b], isem[2 * b]))
            ic.append(pltpu.async_copy(
                dst_hbm.at[pl.ds(base + b * CH, CH)], didx[b],
                isem[2 * b + 1]))
        gc = []
        for b in range(NBUF):
            ic[2 * b].wait()
            gc.append(pltpu.async_copy(m_hbm.at[sidx[b]], rows[b], gsem[b]))
        for b in range(NBUF):
            gc[b].wait()
            ic[2 * b + 1].wait()
            pltpu.sync_copy(rows[b], acc_sh.at[didx[b]], add=True)
        return carry
    lax.fori_loop(0, NCHUNK_P // NBUF, grp, 0)
    plsc.subcore_barrier()

    pltpu.sync_copy(acc_sh.at[pl.ds(sid * RPS, RPS)],
                    out_hbm.at[cid, pl.ds(sid * RPS, RPS)])


RB = 2000
NBLK = N_NODES // RB


def _tc_pre_body(degp_ref, nf_ref, loc_ref, emb_ref, wu_ref, bu_ref, w0_ref,
                 msc_ref, dinv_ref):
    deg = 1.0 + jnp.sum(degp_ref[...], axis=1, keepdims=True)
    dinv = lax.rsqrt(jnp.maximum(deg, 1.0))
    idx = loc_ref[...].astype(jnp.int32) - 1
    cols = lax.broadcasted_iota(jnp.int32, (RB, N_LOC_PAD), 1)
    onehot = (cols == idx).astype(jnp.float32)
    h0 = nf_ref[...] + jnp.dot(onehot, emb_ref[...],
                               preferred_element_type=jnp.float32)
    h0 = jnp.dot(h0, wu_ref[...],
                 preferred_element_type=jnp.float32) + bu_ref[...]
    msc_ref[...] = jnp.dot(h0, w0_ref[...],
                           preferred_element_type=jnp.float32) * dinv
    dinv_ref[...] = dinv


def _tc_pre(deg_parts_t, nf, loc, emb_pad, w_upd, b_upd, w0):
    return pl.pallas_call(
        _tc_pre_body,
        grid=(NBLK,),
        in_specs=[
            pl.BlockSpec((RB, NW), lambda i: (i, 0)),
            pl.BlockSpec((RB, D), lambda i: (i, 0)),
            pl.BlockSpec((RB, 1), lambda i: (i, 0)),
            pl.BlockSpec((N_LOC_PAD, D), lambda i: (0, 0)),
            pl.BlockSpec((D, D), lambda i: (0, 0)),
            pl.BlockSpec((1, D), lambda i: (0, 0)),
            pl.BlockSpec((D, D), lambda i: (0, 0)),
        ],
        out_specs=[
            pl.BlockSpec((RB, D), lambda i: (i, 0)),
            pl.BlockSpec((RB, 1), lambda i: (i, 0)),
        ],
        out_shape=[
            jax.ShapeDtypeStruct((N_NODES, D), jnp.float32),
            jax.ShapeDtypeStruct((N_NODES, 1), jnp.float32),
        ],
    )(deg_parts_t, nf, loc, emb_pad, w_upd, b_upd, w0)


def _bn_lrelu(t, gamma, beta, mean, var):
    t = jnp.where(t > 0, t, 0.2 * t)
    scale = gamma * lax.rsqrt(var + BN_EPS)
    return (t - mean) * scale + beta


def _tc_mid_body(sp_ref, msc_ref, dinv_ref, b_ref, g_ref, be_ref, mu_ref,
                 va_ref, wn_ref, out_ref):
    dinv = dinv_ref[...]
    agg = (sp_ref[0] + sp_ref[1] + msc_ref[...]) * dinv + b_ref[...]
    h = _bn_lrelu(agg, g_ref[...], be_ref[...], mu_ref[...], va_ref[...])
    out_ref[...] = jnp.dot(h, wn_ref[...],
                           preferred_element_type=jnp.float32) * dinv


def _tc_mid(s_parts, msc, dinv, b, gamma, beta, mean, var, w_next):
    return pl.pallas_call(
        _tc_mid_body,
        grid=(NBLK,),
        in_specs=[
            pl.BlockSpec((NC, RB, D), lambda i: (0, i, 0)),
            pl.BlockSpec((RB, D), lambda i: (i, 0)),
            pl.BlockSpec((RB, 1), lambda i: (i, 0)),
            pl.BlockSpec((1, D), lambda i: (0, 0)),
            pl.BlockSpec((1, D), lambda i: (0, 0)),
            pl.BlockSpec((1, D), lambda i: (0, 0)),
            pl.BlockSpec((1, D), lambda i: (0, 0)),
            pl.BlockSpec((1, D), lambda i: (0, 0)),
            pl.BlockSpec((D, D), lambda i: (0, 0)),
        ],
        out_specs=pl.BlockSpec((RB, D), lambda i: (i, 0)),
        out_shape=jax.ShapeDtypeStruct((N_NODES, D), jnp.float32),
    )(s_parts, msc, dinv, b, gamma, beta, mean, var, w_next)


def _tc_fin_body(sp_ref, msc_ref, dinv_ref, b_ref, g_ref, be_ref, mu_ref,
                 va_ref, w1_ref, b1_ref, w2_ref, b2_ref, out_ref, acc_ref):
    i = pl.program_id(0)
    agg = (sp_ref[0] + sp_ref[1] + msc_ref[...]) * dinv_ref[...] + b_ref[...]
    h = _bn_lrelu(agg, g_ref[...], be_ref[...], mu_ref[...], va_ref[...])
    colsum = jnp.sum(h, axis=0, keepdims=True)

    @pl.when(i == 0)
    def _():
        acc_ref[...] = colsum

    @pl.when(i > 0)
    def _():
        acc_ref[...] = acc_ref[...] + colsum

    @pl.when(i == NBLK - 1)
    def _():
        gf = acc_ref[...]
        t1 = jnp.dot(gf, w1_ref[...],
                     preferred_element_type=jnp.float32) + b1_ref[...]
        t1 = jnp.where(t1 > 0, t1, 0.2 * t1)
        out_ref[...] = jnp.dot(t1, w2_ref[...],
                               preferred_element_type=jnp.float32) + b2_ref[...]


def _tc_fin(s_parts, msc, dinv, b, gamma, beta, mean, var, w1, b1, w2, b2):
    return pl.pallas_call(
        _tc_fin_body,
        grid=(NBLK,),
        in_specs=[
            pl.BlockSpec((NC, RB, D), lambda i: (0, i, 0)),
            pl.BlockSpec((RB, D), lambda i: (i, 0)),
            pl.BlockSpec((RB, 1), lambda i: (i, 0)),
            pl.BlockSpec((1, D), lambda i: (0, 0)),
            pl.BlockSpec((1, D), lambda i: (0, 0)),
            pl.BlockSpec((1, D), lambda i: (0, 0)),
            pl.BlockSpec((1, D), lambda i: (0, 0)),
            pl.BlockSpec((1, D), lambda i: (0, 0)),
            pl.BlockSpec((D, 64), lambda i: (0, 0)),
            pl.BlockSpec((1, 64), lambda i: (0, 0)),
            pl.BlockSpec((64, 2), lambda i: (0, 0)),
            pl.BlockSpec((1, 2), lambda i: (0, 0)),
        ],
        out_specs=pl.BlockSpec((1, 2), lambda i: (0, 0)),
        out_shape=jax.ShapeDtypeStruct((1, 2), jnp.float32),
        scratch_shapes=[pltpu.VMEM((1, D), jnp.float32)],
    )(s_parts, msc, dinv, b, gamma, beta, mean, var, w1, b1, w2, b2)


def kernel(x, edge_index, params):
    src = edge_index[0].astype(jnp.int32)
    dst = edge_index[1].astype(jnp.int32)
    pad = EPW_P - EPW
    src3 = jnp.pad(src.reshape(NW, EPW),
                   ((0, 0), (0, pad))).reshape(NW * EPW_P)
    dst3 = jnp.pad(dst.reshape(NW, EPW), ((0, 0), (0, pad)),
                   constant_values=NPAD - 1).reshape(NW * EPW_P)
    nf = x[:, :D]
    loc = x[:, D:D + 1]
    emb_pad = jnp.zeros((N_LOC_PAD, D), jnp.float32).at[:200].set(params['emb'])

    deg_parts = _deg_kernel(dst)
    deg_parts_t = deg_parts.T

    msc, dinv = _tc_pre(deg_parts_t, nf, loc, emb_pad,
                        params['W_upd'], params['b_upd'].reshape(1, D),
                        params['conv0_W'])

    for i in range(2):
        s_parts = _agg_kernel(msc, src3, dst3)
        msc = _tc_mid(s_parts, msc, dinv,
                      params[f'conv{i}_b'].reshape(1, D),
                      params[f'bn{i}_gamma'].reshape(1, D),
                      params[f'bn{i}_beta'].reshape(1, D),
                      params[f'bn{i}_mean'].reshape(1, D),
                      params[f'bn{i}_var'].reshape(1, D),
                      params[f'conv{i + 1}_W'])

    s_parts = _agg_kernel(msc, src3, dst3)
    out = _tc_fin(s_parts, msc, dinv,
                  params['conv2_b'].reshape(1, D),
                  params['bn2_gamma'].reshape(1, D),
                  params['bn2_beta'].reshape(1, D),
                  params['bn2_mean'].reshape(1, D),
                  params['bn2_var'].reshape(1, D),
                  params['W_lin1'], params['b_lin1'].reshape(1, 64),
                  params['W_lin2'], params['b_lin2'].reshape(1, 2))
    return out


# speedup vs baseline: 2.3996x; 2.3996x over previous
import functools

import jax
import jax.numpy as jnp
from jax import lax
from jax.experimental import pallas as pl
from jax.experimental.pallas import tpu as pltpu
from jax.experimental.pallas import tpu_sc as plsc

N_NODES = 10000
N_EDGES = 320000
D = 128
N_LOC_PAD = 256
BN_EPS = 1e-5

NC, NS, L = 2, 16, 16
NW = NC * NS
EPW = N_EDGES // NW
CH = 80
NCHUNK = EPW // CH
NPAD = 10240
RPS = NPAD // NS

_sc_mesh = plsc.VectorSubcoreMesh(core_axis_name="c", subcore_axis_name="s",
                                  num_cores=NC, num_subcores=NS)
_sc_params = pltpu.CompilerParams(needs_layout_passes=False)


DEG_CH = 2000


@functools.partial(
    pl.kernel,
    out_type=jax.ShapeDtypeStruct((NW, N_NODES), jnp.float32),
    mesh=_sc_mesh,
    compiler_params=_sc_params,
    scratch_types=[pltpu.VMEM((DEG_CH,), jnp.int32),
                   pltpu.VMEM((N_NODES,), jnp.float32)])
def _deg_kernel(dst_hbm, out_hbm, idx_v, hist_v):
    wid = lax.axis_index("s") * NC + lax.axis_index("c")

    def zero_body(i, carry):
        hist_v[pl.ds(i * L, L)] = jnp.zeros((L,), jnp.float32)
        return carry
    lax.fori_loop(0, N_NODES // L, zero_body, 0)

    def chunk_body(c, carry):
        pltpu.sync_copy(dst_hbm.at[pl.ds(wid * EPW + c * DEG_CH, DEG_CH)],
                        idx_v)

        def grp(j, carry2):
            ids = idx_v[pl.ds(j * L, L)]
            plsc.addupdate_scatter(hist_v, [ids], jnp.ones((L,), jnp.float32))
            return carry2
        lax.fori_loop(0, DEG_CH // L, grp, 0)
        return carry
    lax.fori_loop(0, EPW // DEG_CH, chunk_body, 0)
    pltpu.sync_copy(hist_v, out_hbm.at[wid])


EPW_P = 10080
NCHUNK_P = EPW_P // CH
NBUF = 3


@functools.partial(
    pl.kernel,
    out_type=jax.ShapeDtypeStruct((NC, NPAD, D), jnp.float32),
    mesh=_sc_mesh,
    compiler_params=_sc_params,
    scratch_types=[pltpu.VMEM((CH,), jnp.int32)] * (2 * NBUF)
                  + [pltpu.VMEM((CH, D), jnp.float32)] * NBUF
                  + [pltpu.VMEM_SHARED((NPAD, D), jnp.float32)]
                  + [pltpu.SemaphoreType.DMA] * (3 * NBUF))
def _agg_kernel(m_hbm, src_hbm, dst_hbm, out_hbm, *rest):
    sidx = rest[0:NBUF]
    didx = rest[NBUF:2 * NBUF]
    rows = rest[2 * NBUF:3 * NBUF]
    acc_sh = rest[3 * NBUF]
    isem = rest[3 * NBUF + 1:3 * NBUF + 1 + 2 * NBUF]
    gsem = rest[3 * NBUF + 1 + 2 * NBUF:]
    cid = lax.axis_index("c")
    sid = lax.axis_index("s")
    wid = sid * NC + cid

    def zb(i, carry):
        r = i // (D // L)
        col = (i % (D // L)) * L
        rows[0][r, pl.ds(col, L)] = jnp.zeros((L,), jnp.float32)
        return carry
    lax.fori_loop(0, CH * (D // L), zb, 0)

    def zcopy(k, carry):
        pltpu.sync_copy(rows[0], acc_sh.at[pl.ds(sid * RPS + k * CH, CH)])
        return carry
    lax.fori_loop(0, RPS // CH, zcopy, 0)
    plsc.subcore_barrier()

    def grp(g, carry):
        base = wid * EPW_P + (2 * g) * CH
        ic = []
        for b in range(NBUF):
            ic.append(pltpu.async_copy(
                src_hbm.at[pl.ds(base + b * CH, CH)], sidx[b], isem[2 * b]))
            ic.append(pltpu.async_copy(
                dst_hbm.at[pl.ds(base + b * CH, CH)], didx[b],
                isem[2 * b + 1]))
        gc = []
        for b in range(NBUF):
            ic[2 * b].wait()
            gc.append(pltpu.async_copy(m_hbm.at[sidx[b]], rows[b], gsem[b]))
        for b in range(NBUF):
            gc[b].wait()
            ic[2 * b + 1].wait()
            pltpu.sync_copy(rows[b], acc_sh.at[didx[b]], add=True)
        return carry
    lax.fori_loop(0, NCHUNK_P // NBUF, grp, 0)
    plsc.subcore_barrier()

    pltpu.sync_copy(acc_sh.at[pl.ds(sid * RPS, RPS)],
                    out_hbm.at[cid, pl.ds(sid * RPS, RPS)])


RB = 2000
NBLK = N_NODES // RB


def _tc_pre_body(degp_ref, nf_ref, loc_ref, emb_ref, wu_ref, bu_ref, w0_ref,
                 msc_ref, dinv_ref):
    deg = 1.0 + jnp.sum(degp_ref[...], axis=1, keepdims=True)
    dinv = lax.rsqrt(jnp.maximum(deg, 1.0))
    idx = loc_ref[...].astype(jnp.int32) - 1
    cols = lax.broadcasted_iota(jnp.int32, (RB, N_LOC_PAD), 1)
    onehot = (cols == idx).astype(jnp.float32)
    h0 = nf_ref[...] + jnp.dot(onehot, emb_ref[...],
                               preferred_element_type=jnp.float32)
    h0 = jnp.dot(h0, wu_ref[...],
                 preferred_element_type=jnp.float32) + bu_ref[...]
    msc_ref[...] = jnp.dot(h0, w0_ref[...],
                           preferred_element_type=jnp.float32) * dinv
    dinv_ref[...] = dinv


def _tc_pre(deg_parts_t, nf, loc, emb_pad, w_upd, b_upd, w0):
    return pl.pallas_call(
        _tc_pre_body,
        grid=(NBLK,),
        in_specs=[
            pl.BlockSpec((RB, NW), lambda i: (i, 0)),
            pl.BlockSpec((RB, D), lambda i: (i, 0)),
            pl.BlockSpec((RB, 1), lambda i: (i, 0)),
            pl.BlockSpec((N_LOC_PAD, D), lambda i: (0, 0)),
            pl.BlockSpec((D, D), lambda i: (0, 0)),
            pl.BlockSpec((1, D), lambda i: (0, 0)),
            pl.BlockSpec((D, D), lambda i: (0, 0)),
        ],
        out_specs=[
            pl.BlockSpec((RB, D), lambda i: (i, 0)),
            pl.BlockSpec((RB, 1), lambda i: (i, 0)),
        ],
        out_shape=[
            jax.ShapeDtypeStruct((N_NODES, D), jnp.float32),
            jax.ShapeDtypeStruct((N_NODES, 1), jnp.float32),
        ],
    )(deg_parts_t, nf, loc, emb_pad, w_upd, b_upd, w0)


def _bn_lrelu(t, gamma, beta, mean, var):
    t = jnp.where(t > 0, t, 0.2 * t)
    scale = gamma * lax.rsqrt(var + BN_EPS)
    return (t - mean) * scale + beta


def _tc_mid_body(sp_ref, msc_ref, dinv_ref, b_ref, g_ref, be_ref, mu_ref,
                 va_ref, wn_ref, out_ref):
    dinv = dinv_ref[...]
    agg = (sp_ref[0] + sp_ref[1] + msc_ref[...]) * dinv + b_ref[...]
    h = _bn_lrelu(agg, g_ref[...], be_ref[...], mu_ref[...], va_ref[...])
    out_ref[...] = jnp.dot(h, wn_ref[...],
                           preferred_element_type=jnp.float32) * dinv


def _tc_mid(s_parts, msc, dinv, b, gamma, beta, mean, var, w_next):
    return pl.pallas_call(
        _tc_mid_body,
        grid=(NBLK,),
        in_specs=[
            pl.BlockSpec((NC, RB, D), lambda i: (0, i, 0)),
            pl.BlockSpec((RB, D), lambda i: (i, 0)),
            pl.BlockSpec((RB, 1), lambda i: (i, 0)),
            pl.BlockSpec((1, D), lambda i: (0, 0)),
            pl.BlockSpec((1, D), lambda i: (0, 0)),
            pl.BlockSpec((1, D), lambda i: (0, 0)),
            pl.BlockSpec((1, D), lambda i: (0, 0)),
            pl.BlockSpec((1, D), lambda i: (0, 0)),
            pl.BlockSpec((D, D), lambda i: (0, 0)),
        ],
        out_specs=pl.BlockSpec((RB, D), lambda i: (i, 0)),
        out_shape=jax.ShapeDtypeStruct((N_NODES, D), jnp.float32),
    )(s_parts, msc, dinv, b, gamma, beta, mean, var, w_next)


def _tc_fin_body(sp_ref, msc_ref, dinv_ref, b_ref, g_ref, be_ref, mu_ref,
                 va_ref, w1_ref, b1_ref, w2_ref, b2_ref, out_ref, acc_ref):
    i = pl.program_id(0)
    agg = (sp_ref[0] + sp_ref[1] + msc_ref[...]) * dinv_ref[...] + b_ref[...]
    h = _bn_lrelu(agg, g_ref[...], be_ref[...], mu_ref[...], va_ref[...])
    colsum = jnp.sum(h, axis=0, keepdims=True)

    @pl.when(i == 0)
    def _():
        acc_ref[...] = colsum

    @pl.when(i > 0)
    def _():
        acc_ref[...] = acc_ref[...] + colsum

    @pl.when(i == NBLK - 1)
    def _():
        gf = acc_ref[...]
        t1 = jnp.dot(gf, w1_ref[...],
                     preferred_element_type=jnp.float32) + b1_ref[...]
        t1 = jnp.where(t1 > 0, t1, 0.2 * t1)
        out_ref[...] = jnp.dot(t1, w2_ref[...],
                               preferred_element_type=jnp.float32) + b2_ref[...]


def _tc_fin(s_parts, msc, dinv, b, gamma, beta, mean, var, w1, b1, w2, b2):
    return pl.pallas_call(
        _tc_fin_body,
        grid=(NBLK,),
        in_specs=[
            pl.BlockSpec((NC, RB, D), lambda i: (0, i, 0)),
            pl.BlockSpec((RB, D), lambda i: (i, 0)),
            pl.BlockSpec((RB, 1), lambda i: (i, 0)),
            pl.BlockSpec((1, D), lambda i: (0, 0)),
            pl.BlockSpec((1, D), lambda i: (0, 0)),
            pl.BlockSpec((1, D), lambda i: (0, 0)),
            pl.BlockSpec((1, D), lambda i: (0, 0)),
            pl.BlockSpec((1, D), lambda i: (0, 0)),
            pl.BlockSpec((D, 64), lambda i: (0, 0)),
            pl.BlockSpec((1, 64), lambda i: (0, 0)),
            pl.BlockSpec((64, 2), lambda i: (0, 0)),
            pl.BlockSpec((1, 2), lambda i: (0, 0)),
        ],
        out_specs=pl.BlockSpec((1, 2), lambda i: (0, 0)),
        out_shape=jax.ShapeDtypeStruct((1, 2), jnp.float32),
        scratch_shapes=[pltpu.VMEM((1, D), jnp.float32)],
    )(s_parts, msc, dinv, b, gamma, beta, mean, var, w1, b1, w2, b2)


def kernel(x, edge_index, params):
    src = edge_index[0].astype(jnp.int32)
    dst = edge_index[1].astype(jnp.int32)
    pad = EPW_P - EPW
    src3 = jnp.pad(src.reshape(NW, EPW),
                   ((0, 0), (0, pad))).reshape(NW * EPW_P)
    dst3 = jnp.pad(dst.reshape(NW, EPW), ((0, 0), (0, pad)),
                   constant_values=NPAD - 1).reshape(NW * EPW_P)
    nf = x[:, :D]
    loc = x[:, D:D + 1]
    emb_pad = jnp.zeros((N_LOC_PAD, D), jnp.float32).at[:200].set(params['emb'])

    deg_parts = _deg_kernel(dst)
    deg_parts_t = deg_parts.T

    msc, dinv = _tc_pre(deg_parts_t, nf, loc, emb_pad,
                        params['W_upd'], params['b_upd'].reshape(1, D),
                        params['conv0_W'])

    for i in range(2):
        s_parts = _agg_kernel(msc, src3, dst3)
        msc = _tc_mid(s_parts, msc, dinv,
                      params[f'conv{i}_b'].reshape(1, D),
                      params[f'bn{i}_gamma'].reshape(1, D),
                      params[f'bn{i}_beta'].reshape(1, D),
                      params[f'bn{i}_mean'].reshape(1, D),
                      params[f'bn{i}_var'].reshape(1, D),
                      params[f'conv{i + 1}_W'])

    s_parts = _agg_kernel(msc, src3, dst3)
    out = _tc_fin(s_parts, msc, dinv,
                  params['conv2_b'].reshape(1, D),
                  params['bn2_gamma'].reshape(1, D),
                  params['bn2_beta'].reshape(1, D),
                  params['bn2_mean'].reshape(1, D),
                  params['bn2_var'].reshape(1, D),
                  params['W_lin1'], params['b_lin1'].reshape(1, 64),
                  params['W_lin2'], params['b_lin2'].reshape(1, 2))
    return out
